# baseline (speedup 1.0000x reference)
import functools

import jax
import jax.numpy as jnp
from jax import lax
from jax.experimental import pallas as pl
from jax.experimental.pallas import tpu as pltpu
from jax.experimental.pallas import tpu_sc as plsc

_LANES = 16


@functools.lru_cache(maxsize=None)
def _make_kernel(B, L, V, D):
    info = plsc.get_sparse_core_info()
    NC, NS = info.num_cores, info.num_subcores
    NW = NC * NS
    assert L % NW == 0 and D % _LANES == 0
    LW = L // NW
    C = min(32, LW)
    assert LW % C == 0 and C % 8 == 0
    n_chunks = LW // C
    n_vec = D // _LANES
    steps = [(ci, b) for ci in range(n_chunks) for b in range(B)]

    mesh = plsc.VectorSubcoreMesh(core_axis_name="c", subcore_axis_name="s")

    @functools.partial(
        pl.kernel,
        mesh=mesh,
        out_type=jax.ShapeDtypeStruct((B, L, D), jnp.float32),
        scratch_types=[
            pltpu.VMEM((B, LW), jnp.int32),
            pltpu.VMEM((2, C, D), jnp.float32),
            pltpu.VMEM((2, C, D), jnp.float32),
            pltpu.SemaphoreType.DMA,
            pltpu.SemaphoreType.DMA,
            pltpu.SemaphoreType.DMA,
        ],
    )
    def emb(x_hbm, table_hbm, enc_hbm, out_hbm,
            idx_v, enc_v, rows_v, gsem, esem, isem):
        wid = lax.axis_index("s") * NC + lax.axis_index("c")
        l0 = wid * LW

        def enc_descr(ci):
            return pltpu.make_async_copy(
                enc_hbm.at[pl.ds(l0 + ci * C, C)], enc_v.at[ci % 2], esem
            )

        enc_descr(0).start()
        for b in range(B):
            pltpu.async_copy(x_hbm.at[b, pl.ds(l0, LW)], idx_v.at[b], isem)
        for b in range(B):
            pltpu.make_async_copy(
                x_hbm.at[b, pl.ds(l0, LW)], idx_v.at[b], isem
            ).wait()

        def fire(ci, b, slot):
            pltpu.async_copy(
                table_hbm.at[idx_v.at[b, pl.ds(ci * C, C)]],
                rows_v.at[slot],
                gsem,
            )

        fire(0, 0, 0)

        for t, (ci, b) in enumerate(steps):
            slot = t % 2
            if t + 1 < len(steps):
                fire(steps[t + 1][0], steps[t + 1][1], (t + 1) % 2)
            if b == 0:
                if ci + 1 < n_chunks:
                    enc_descr(ci + 1).start()
                enc_descr(ci).wait()
            ebuf = ci % 2
            pltpu.make_async_copy(
                table_hbm.at[idx_v.at[b, pl.ds(ci * C, C)]],
                rows_v.at[slot],
                gsem,
            ).wait()

            def row_body(r, _, slot=slot, ebuf=ebuf):
                for j in range(n_vec):
                    sl = pl.ds(j * _LANES, _LANES)
                    plsc.addupdate(rows_v.at[slot, r, sl], enc_v[ebuf, r, sl])
                return 0

            lax.fori_loop(0, C, row_body, 0)
            pltpu.sync_copy(rows_v.at[slot], out_hbm.at[b, pl.ds(l0 + ci * C, C)])

    return emb


def kernel(x, table, enc):
    B, L = x.shape
    V, D = table.shape
    emb = _make_kernel(B, L, V, D)
    return emb(x.astype(jnp.int32), table, enc)


# speedup vs baseline: 1.3928x; 1.3928x over previous
import functools

import jax
import jax.numpy as jnp
from jax import lax
from jax.experimental import pallas as pl
from jax.experimental.pallas import tpu as pltpu
from jax.experimental.pallas import tpu_sc as plsc

_LANES = 16


@functools.lru_cache(maxsize=None)
def _make_kernel(B, L, V, D):
    info = plsc.get_sparse_core_info()
    NC, NS = info.num_cores, info.num_subcores
    NW = NC * NS
    assert L % NW == 0 and D % _LANES == 0
    LW = L // NW
    C = min(16, LW)
    assert LW % C == 0 and C % 8 == 0
    n_chunks = LW // C
    n_vec = D // _LANES

    mesh = plsc.VectorSubcoreMesh(core_axis_name="c", subcore_axis_name="s")

    @functools.partial(
        pl.kernel,
        mesh=mesh,
        out_type=jax.ShapeDtypeStruct((B, L, D), jnp.float32),
        scratch_types=[
            pltpu.VMEM((B, LW), jnp.int32),
            pltpu.VMEM((C, D), jnp.float32),
            pltpu.VMEM((2, B, C, D), jnp.float32),
            pltpu.SemaphoreType.DMA,
            pltpu.SemaphoreType.DMA,
            pltpu.SemaphoreType.DMA,
        ],
    )
    def emb(x_hbm, table_hbm, enc_hbm, out_hbm,
            idx_v, enc_v, rows_v, gsem, esem, isem):
        wid = lax.axis_index("s") * NC + lax.axis_index("c")
        l0 = wid * LW

        for b in range(B):
            pltpu.async_copy(x_hbm.at[b, pl.ds(l0, LW)], idx_v.at[b], isem)
        for b in range(B):
            pltpu.make_async_copy(
                x_hbm.at[b, pl.ds(l0, LW)], idx_v.at[b], isem
            ).wait()

        def fire(ci, slot):
            for b in range(B):
                pltpu.async_copy(
                    table_hbm.at[idx_v.at[b, pl.ds(ci * C, C)]],
                    rows_v.at[slot, b],
                    gsem,
                )

        def enc_descr(ci):
            return pltpu.make_async_copy(
                enc_hbm.at[pl.ds(l0 + ci * C, C)], enc_v, esem
            )

        enc_descr(0).start()
        fire(0, 0)

        for ci in range(n_chunks):
            slot = ci % 2
            if ci + 1 < n_chunks:
                fire(ci + 1, 1 - slot)
            enc_descr(ci).wait()
            for b in range(B):
                pltpu.make_async_copy(
                    table_hbm.at[idx_v.at[b, pl.ds(ci * C, C)]],
                    rows_v.at[slot, b],
                    gsem,
                ).wait()

            def row_body(r, _, slot=slot):
                for j in range(n_vec):
                    sl = pl.ds(j * _LANES, _LANES)
                    e = enc_v[r, sl]
                    for b in range(B):
                        plsc.addupdate(rows_v.at[slot, b, r, sl], e)
                return 0

            lax.fori_loop(0, C, row_body, 0)
            if ci + 1 < n_chunks:
                enc_descr(ci + 1).start()
            for b in range(B):
                pltpu.sync_copy(
                    rows_v.at[slot, b], out_hbm.at[b, pl.ds(l0 + ci * C, C)]
                )

    return emb


def kernel(x, table, enc):
    B, L = x.shape
    V, D = table.shape
    emb = _make_kernel(B, L, V, D)
    return emb(x.astype(jnp.int32), table, enc)
